# baseline (speedup 1.0000x reference)
import functools

import jax
import jax.numpy as jnp
import numpy as np
from jax import lax
from jax.experimental import pallas as pl
from jax.experimental.pallas import tpu as pltpu
from jax.experimental.pallas import tpu_sc as plsc

_RADII = (0.2, 0.4)
_NSAMP = (16, 32)
_NC, _NSUB, _L = 2, 16, 16
_NW = _NC * _NSUB


@functools.lru_cache(maxsize=None)
def _sc_group_kernel(B, N, S, C):
    n_centers = B * S
    cpt = n_centers // _NW
    tiles_per_b = S // cpt
    ns1, ns2 = _NSAMP
    r1sq = np.float32(_RADII[0] * _RADII[0])
    r2sq = np.float32(_RADII[1] * _RADII[1])
    nch = N // _L
    p1t, p2t = cpt * ns1, cpt * ns2
    g1rows, g2rows = p1t // 128, p2t // 128

    mesh = plsc.VectorSubcoreMesh(
        core_axis_name="c", subcore_axis_name="s",
        num_cores=_NC, num_subcores=_NSUB)

    out_type = (
        jax.ShapeDtypeStruct((n_centers * 3,), jnp.float32),
        jax.ShapeDtypeStruct((n_centers * ns1, 8), jnp.float32),
        jax.ShapeDtypeStruct((n_centers * ns2, 8), jnp.float32),
        jax.ShapeDtypeStruct((n_centers * ns1, C), jnp.float32),
        jax.ShapeDtypeStruct((n_centers * ns2, C), jnp.float32),
    )
    scratch = [
        pltpu.VMEM((N,), jnp.float32),
        pltpu.VMEM((N,), jnp.float32),
        pltpu.VMEM((N,), jnp.float32),
        pltpu.VMEM((cpt,), jnp.int32),
        pltpu.VMEM((cpt,), jnp.float32),
        pltpu.VMEM((cpt,), jnp.float32),
        pltpu.VMEM((cpt,), jnp.float32),
        pltpu.VMEM((cpt * 3,), jnp.float32),
        pltpu.VMEM((ns1 + _L,), jnp.int32),
        pltpu.VMEM((ns2 + _L,), jnp.int32),
        pltpu.VMEM((p1t, 8), jnp.float32),
        pltpu.VMEM((p2t, 8), jnp.float32),
        pltpu.VMEM((g1rows, 128), jnp.int32),
        pltpu.VMEM((g2rows, 128), jnp.int32),
        pltpu.VMEM((128, C), jnp.float32),
        pltpu.VMEM((128, C), jnp.float32),
        pltpu.VMEM((N,), jnp.float32),
        pltpu.SemaphoreType.DMA,
    ]

    @functools.partial(
        pl.kernel, out_type=out_type, mesh=mesh, scratch_types=scratch,
        compiler_params=pltpu.CompilerParams(
            needs_layout_passes=False, use_tc_tiling_on_sc=False))
    def kern(xyz_hbm, inds_hbm, feat_hbm,
             nxyz_hbm, dxyz1_hbm, dxyz2_hbm, feat1_hbm, feat2_hbm,
             x_v, y_v, z_v, inds_v, cx_v, cy_v, cz_v, nxyz_v,
             buf1, buf2, dxyz1_v, dxyz2_v, gidx1_v, gidx2_v,
             rows_a, rows_b, bb_v, semg):
        wid = lax.axis_index("s") * _NC + lax.axis_index("c")
        b = wid // tiles_per_b
        boff = b * N

        pltpu.sync_copy(xyz_hbm.at[b * 3 + 0], x_v)
        pltpu.sync_copy(xyz_hbm.at[b * 3 + 1], y_v)
        pltpu.sync_copy(xyz_hbm.at[b * 3 + 2], z_v)
        pltpu.sync_copy(inds_hbm.at[pl.ds(wid * cpt, cpt)], inds_v)

        lane = lax.iota(jnp.int32, _L)
        zero16 = jnp.zeros((_L,), jnp.int32)

        for c in range(cpt // _L):
            iv = inds_v[pl.ds(c * _L, _L)]
            gx = plsc.load_gather(x_v, [iv])
            gy = plsc.load_gather(y_v, [iv])
            gz = plsc.load_gather(z_v, [iv])
            cx_v[pl.ds(c * _L, _L)] = gx
            cy_v[pl.ds(c * _L, _L)] = gy
            cz_v[pl.ds(c * _L, _L)] = gz
            pos = (lane + c * _L) * 3
            plsc.store_scatter(nxyz_v, [pos], gx)
            plsc.store_scatter(nxyz_v, [pos + 1], gy)
            plsc.store_scatter(nxyz_v, [pos + 2], gz)
        pltpu.sync_copy(nxyz_v, nxyz_hbm.at[pl.ds(wid * cpt * 3, cpt * 3)])

        def precompute_bb(j, carry):
            base = j * _L
            xi = x_v[pl.ds(base, _L)]
            yi = y_v[pl.ds(base, _L)]
            zi = z_v[pl.ds(base, _L)]
            bb_v[pl.ds(base, _L)] = xi * xi + yi * yi + zi * zi
            return carry

        lax.fori_loop(0, nch, precompute_bb, 0)

        def bf16r(x):
            u = plsc.bitcast(x, jnp.uint32)
            u = (u + jnp.uint32(0x7FFF) + ((u >> jnp.uint32(16)) & jnp.uint32(1)))
            return plsc.bitcast(u & jnp.uint32(0xFFFF0000), jnp.float32)

        def per_center(i, carry):
            ci = jnp.full((_L,), i, jnp.int32)
            cx = plsc.load_gather(cx_v, [ci])
            cy = plsc.load_gather(cy_v, [ci])
            cz = plsc.load_gather(cz_v, [ci])
            aa = cx * cx + cy * cy + cz * cz
            cxr, cyr, czr = bf16r(cx), bf16r(cy), bf16r(cz)

            def cond(st):
                j, c1v, c2v = st
                return (j < nch) & jnp.any((c1v < ns1) | (c2v < ns2))

            def body(st):
                j, c1v, c2v = st
                base = j * _L
                xi = x_v[pl.ds(base, _L)]
                yi = y_v[pl.ds(base, _L)]
                zi = z_v[pl.ds(base, _L)]
                bb = bb_v[pl.ds(base, _L)]
                ab = cxr * bf16r(xi) + cyr * bf16r(yi) + czr * bf16r(zi)
                d = (aa + bb) - 2.0 * ab
                pidx = lane + base
                w1 = (d <= r1sq) & (c1v < ns1)
                w2 = (d <= r2sq) & (c2v < ns2)
                pos1 = plsc.cumsum(w1.astype(jnp.int32))
                pos2 = plsc.cumsum(w2.astype(jnp.int32))
                plsc.store_scatter(buf1, [c1v + pos1 - 1], pidx, mask=w1)
                plsc.store_scatter(buf2, [c2v + pos2 - 1], pidx, mask=w2)
                c1v = c1v + plsc.all_reduce_population_count(w1)
                c2v = c2v + plsc.all_reduce_population_count(w2)
                return j + 1, c1v, c2v

            zero_cnt = jnp.zeros((_L,), jnp.int32)
            _, c1, c2 = lax.while_loop(cond, body, (0, zero_cnt, zero_cnt))

            fi1 = plsc.load_gather(buf1, [zero16])
            safe1 = jnp.where(c1 > 0, fi1, zero16)
            v1 = buf1[0:_L]
            idx1 = jnp.where(lane < c1, v1, safe1)
            gx = plsc.load_gather(x_v, [idx1]) - cx
            gy = plsc.load_gather(y_v, [idx1]) - cy
            gz = plsc.load_gather(z_v, [idx1]) - cz
            row = i * ns1 + lane
            plsc.store_scatter(dxyz1_v, [row, zero16], gx)
            plsc.store_scatter(dxyz1_v, [row, zero16 + 1], gy)
            plsc.store_scatter(dxyz1_v, [row, zero16 + 2], gz)
            flat1 = i * ns1
            gidx1_v[flat1 // 128, pl.ds(flat1 % 128, _L)] = idx1 + boff

            fi2 = plsc.load_gather(buf2, [zero16])
            safe2 = jnp.where(c2 > 0, fi2, zero16)
            for h in range(ns2 // _L):
                slot = lane + h * _L
                v2 = buf2[h * _L:(h + 1) * _L]
                idx2 = jnp.where(slot < c2, v2, safe2)
                gx = plsc.load_gather(x_v, [idx2]) - cx
                gy = plsc.load_gather(y_v, [idx2]) - cy
                gz = plsc.load_gather(z_v, [idx2]) - cz
                row = i * ns2 + slot
                plsc.store_scatter(dxyz2_v, [row, zero16], gx)
                plsc.store_scatter(dxyz2_v, [row, zero16 + 1], gy)
                plsc.store_scatter(dxyz2_v, [row, zero16 + 2], gz)
                flat2 = i * ns2 + h * _L
                gidx2_v[flat2 // 128, pl.ds(flat2 % 128, _L)] = idx2 + boff
            return carry

        lax.fori_loop(0, cpt, per_center, 0)

        pltpu.sync_copy(dxyz1_v, dxyz1_hbm.at[pl.ds(wid * p1t, p1t)])
        pltpu.sync_copy(dxyz2_v, dxyz2_hbm.at[pl.ds(wid * p2t, p2t)])

        for gidx_v, nk, out_hbm, obase in (
                (gidx1_v, g1rows, feat1_hbm, wid * p1t),
                (gidx2_v, g2rows, feat2_hbm, wid * p2t)):
            cp = pltpu.async_copy(feat_hbm.at[gidx_v.at[0]], rows_a, semg)
            for k in range(nk):
                buf = rows_a if k % 2 == 0 else rows_b
                nbuf = rows_b if k % 2 == 0 else rows_a
                cp.wait()
                if k + 1 < nk:
                    cp = pltpu.async_copy(
                        feat_hbm.at[gidx_v.at[k + 1]], nbuf, semg)
                pltpu.sync_copy(buf, out_hbm.at[pl.ds(obase + k * 128, 128)])

    return kern


def _mm_stats_body(feat_ref, dxyz_ref, wf_ref, wx_ref, y_ref, st_ref):
    i = pl.program_id(0)
    x1 = feat_ref[...]
    x2 = dxyz_ref[...]
    col = lax.broadcasted_iota(jnp.int32, x2.shape, 1)
    x2 = jnp.where(col < 3, x2, 0.0)
    y = jnp.dot(x1, wf_ref[...], preferred_element_type=jnp.float32)
    y = y + jnp.dot(x2, wx_ref[...], preferred_element_type=jnp.float32)
    y_ref[...] = y

    @pl.when(i == 0)
    def _init():
        st_ref[...] = jnp.zeros_like(st_ref)

    st_ref[0:1, :] += jnp.sum(y, axis=0, keepdims=True)
    st_ref[1:2, :] += jnp.sum(y * y, axis=0, keepdims=True)


def _mm_stats_call(feat, dxyz, wf, wx, rb):
    p, cin = feat.shape
    c1 = wf.shape[1]
    return pl.pallas_call(
        _mm_stats_body,
        grid=(p // rb,),
        in_specs=[
            pl.BlockSpec((rb, cin), lambda i: (i, 0)),
            pl.BlockSpec((rb, 8), lambda i: (i, 0)),
            pl.BlockSpec((cin, c1), lambda i: (0, 0)),
            pl.BlockSpec((8, c1), lambda i: (0, 0)),
        ],
        out_specs=[
            pl.BlockSpec((rb, c1), lambda i: (i, 0)),
            pl.BlockSpec((8, c1), lambda i: (0, 0)),
        ],
        out_shape=[
            jax.ShapeDtypeStruct((p, c1), jnp.float32),
            jax.ShapeDtypeStruct((8, c1), jnp.float32),
        ],
    )(feat, dxyz, wf, wx)


def _bn_mm_body(y_ref, a_ref, c_ref, w_ref, o_ref, st_ref):
    i = pl.program_id(0)
    a = a_ref[0:1, :]
    c = c_ref[0:1, :]
    x = jnp.maximum(y_ref[...] * a + c, 0.0)
    y = jnp.dot(x, w_ref[...], preferred_element_type=jnp.float32)
    o_ref[...] = y

    @pl.when(i == 0)
    def _init():
        st_ref[...] = jnp.zeros_like(st_ref)

    st_ref[0:1, :] += jnp.sum(y, axis=0, keepdims=True)
    st_ref[1:2, :] += jnp.sum(y * y, axis=0, keepdims=True)


def _bn_mm_call(y1, a, c, w2, rb):
    p, c1 = y1.shape
    c2 = w2.shape[1]
    return pl.pallas_call(
        _bn_mm_body,
        grid=(p // rb,),
        in_specs=[
            pl.BlockSpec((rb, c1), lambda i: (i, 0)),
            pl.BlockSpec((8, c1), lambda i: (0, 0)),
            pl.BlockSpec((8, c1), lambda i: (0, 0)),
            pl.BlockSpec((c1, c2), lambda i: (0, 0)),
        ],
        out_specs=[
            pl.BlockSpec((rb, c2), lambda i: (i, 0)),
            pl.BlockSpec((8, c2), lambda i: (0, 0)),
        ],
        out_shape=[
            jax.ShapeDtypeStruct((p, c2), jnp.float32),
            jax.ShapeDtypeStruct((8, c2), jnp.float32),
        ],
    )(y1, a, c, w2)


def _bn_max_body(y_ref, a_ref, c_ref, o_ref):
    a = jnp.reshape(a_ref[0:1, :], (1, 1, -1))
    c = jnp.reshape(c_ref[0:1, :], (1, 1, -1))
    x = jnp.maximum(y_ref[...] * a + c, 0.0)
    o_ref[...] = jnp.max(x, axis=1)


def _bn_max_call(y3d, a, c, sb):
    ncen, ns, c2 = y3d.shape
    return pl.pallas_call(
        _bn_max_body,
        grid=(ncen // sb,),
        in_specs=[
            pl.BlockSpec((sb, ns, c2), lambda i: (i, 0, 0)),
            pl.BlockSpec((8, c2), lambda i: (0, 0)),
            pl.BlockSpec((8, c2), lambda i: (0, 0)),
        ],
        out_specs=pl.BlockSpec((sb, c2), lambda i: (i, 0)),
        out_shape=jax.ShapeDtypeStruct((ncen, c2), jnp.float32),
    )(y3d, a, c)


def _bn_coeffs(st, count, gamma, beta):
    mu = st[0] / count
    var = st[1] / count - mu * mu
    a = gamma * lax.rsqrt(var + 1e-5)
    c = beta - mu * a
    a8 = jnp.broadcast_to(a[None, :], (8, a.shape[0]))
    c8 = jnp.broadcast_to(c[None, :], (8, c.shape[0]))
    return a8, c8


def kernel(xyz, features, inds, params):
    B, N, _ = xyz.shape
    C = features.shape[1]
    S = inds.shape[1]

    xyz_t = jnp.transpose(xyz, (0, 2, 1)).reshape(B * 3, N)
    inds_flat = inds.reshape(B * S).astype(jnp.int32)
    feat_rows = jnp.transpose(features, (0, 2, 1)).reshape(B * N, C)

    sck = _sc_group_kernel(B, N, S, C)
    nxyz_flat, dxyz1, dxyz2, feat1, feat2 = sck(xyz_t, inds_flat, feat_rows)
    new_xyz = nxyz_flat.reshape(B, S, 3)

    outs = []
    for scale, (ns, dxyz, feat) in enumerate(
            ((_NSAMP[0], dxyz1, feat1), (_NSAMP[1], dxyz2, feat2))):
        (w1, g1, b1), (w2, g2, b2) = params[scale]
        p = B * S * ns
        wf = jnp.transpose(w1[:, 3:])
        wx = jnp.zeros((8, w1.shape[0]), jnp.float32)
        wx = wx.at[0:3].set(jnp.transpose(w1[:, 0:3]))
        y1, st1 = _mm_stats_call(feat, dxyz, wf, wx, rb=4096)
        a1, c1 = _bn_coeffs(st1, p, g1, b1)
        y2, st2 = _bn_mm_call(y1, a1, c1, jnp.transpose(w2), rb=4096)
        a2, c2 = _bn_coeffs(st2, p, g2, b2)
        o = _bn_max_call(y2.reshape(B * S, ns, -1), a2, c2, sb=256)
        outs.append(jnp.transpose(o.reshape(B, S, -1), (0, 2, 1)))

    return new_xyz, jnp.concatenate(outs, axis=1), inds


# speedup vs baseline: 1.0817x; 1.0817x over previous
import functools

import jax
import jax.numpy as jnp
import numpy as np
from jax import lax
from jax.experimental import pallas as pl
from jax.experimental.pallas import tpu as pltpu
from jax.experimental.pallas import tpu_sc as plsc

_RADII = (0.2, 0.4)
_NSAMP = (16, 32)
_NC, _NSUB, _L = 2, 16, 16
_NW = _NC * _NSUB


@functools.lru_cache(maxsize=None)
def _sc_group_kernel(B, N, S, C):
    n_centers = B * S
    cpt = n_centers // _NW
    tiles_per_b = S // cpt
    ns1, ns2 = _NSAMP
    r1sq = np.float32(_RADII[0] * _RADII[0])
    r2sq = np.float32(_RADII[1] * _RADII[1])
    nch = N // _L
    p1t, p2t = cpt * ns1, cpt * ns2
    g1rows, g2rows = p1t // 128, p2t // 128

    mesh = plsc.VectorSubcoreMesh(
        core_axis_name="c", subcore_axis_name="s",
        num_cores=_NC, num_subcores=_NSUB)

    out_type = (
        jax.ShapeDtypeStruct((n_centers * 3,), jnp.float32),
        jax.ShapeDtypeStruct((n_centers * ns1, 8), jnp.float32),
        jax.ShapeDtypeStruct((n_centers * ns2, 8), jnp.float32),
        jax.ShapeDtypeStruct((n_centers * ns1, C), jnp.float32),
        jax.ShapeDtypeStruct((n_centers * ns2, C), jnp.float32),
    )
    scratch = [
        pltpu.VMEM((N,), jnp.float32),
        pltpu.VMEM((N,), jnp.float32),
        pltpu.VMEM((N,), jnp.float32),
        pltpu.VMEM((cpt,), jnp.int32),
        pltpu.VMEM((cpt,), jnp.float32),
        pltpu.VMEM((cpt,), jnp.float32),
        pltpu.VMEM((cpt,), jnp.float32),
        pltpu.VMEM((cpt * 3,), jnp.float32),
        pltpu.VMEM((ns1 + _L,), jnp.int32),
        pltpu.VMEM((ns2 + _L,), jnp.int32),
        pltpu.VMEM((p1t, 8), jnp.float32),
        pltpu.VMEM((p2t, 8), jnp.float32),
        pltpu.VMEM((g1rows, 128), jnp.int32),
        pltpu.VMEM((g2rows, 128), jnp.int32),
        pltpu.VMEM((128, C), jnp.float32),
        pltpu.VMEM((128, C), jnp.float32),
        pltpu.VMEM((N,), jnp.float32),
        pltpu.SemaphoreType.DMA,
    ]

    @functools.partial(
        pl.kernel, out_type=out_type, mesh=mesh, scratch_types=scratch,
        compiler_params=pltpu.CompilerParams(
            needs_layout_passes=False, use_tc_tiling_on_sc=False))
    def kern(xyz_hbm, inds_hbm, feat_hbm,
             nxyz_hbm, dxyz1_hbm, dxyz2_hbm, feat1_hbm, feat2_hbm,
             x_v, y_v, z_v, inds_v, cx_v, cy_v, cz_v, nxyz_v,
             buf1, buf2, dxyz1_v, dxyz2_v, gidx1_v, gidx2_v,
             rows_a, rows_b, bb_v, semg):
        wid = lax.axis_index("s") * _NC + lax.axis_index("c")
        b = wid // tiles_per_b
        boff = b * N

        pltpu.sync_copy(xyz_hbm.at[b * 3 + 0], x_v)
        pltpu.sync_copy(xyz_hbm.at[b * 3 + 1], y_v)
        pltpu.sync_copy(xyz_hbm.at[b * 3 + 2], z_v)
        pltpu.sync_copy(inds_hbm.at[pl.ds(wid * cpt, cpt)], inds_v)

        lane = lax.iota(jnp.int32, _L)
        zero16 = jnp.zeros((_L,), jnp.int32)

        for c in range(cpt // _L):
            iv = inds_v[pl.ds(c * _L, _L)]
            gx = plsc.load_gather(x_v, [iv])
            gy = plsc.load_gather(y_v, [iv])
            gz = plsc.load_gather(z_v, [iv])
            cx_v[pl.ds(c * _L, _L)] = gx
            cy_v[pl.ds(c * _L, _L)] = gy
            cz_v[pl.ds(c * _L, _L)] = gz
            pos = (lane + c * _L) * 3
            plsc.store_scatter(nxyz_v, [pos], gx)
            plsc.store_scatter(nxyz_v, [pos + 1], gy)
            plsc.store_scatter(nxyz_v, [pos + 2], gz)
        pltpu.sync_copy(nxyz_v, nxyz_hbm.at[pl.ds(wid * cpt * 3, cpt * 3)])

        def precompute_bb(j, carry):
            base = j * _L
            xi = x_v[pl.ds(base, _L)]
            yi = y_v[pl.ds(base, _L)]
            zi = z_v[pl.ds(base, _L)]
            bb_v[pl.ds(base, _L)] = xi * xi + yi * yi + zi * zi
            return carry

        lax.fori_loop(0, nch, precompute_bb, 0)

        def bf16r(x):
            u = plsc.bitcast(x, jnp.uint32)
            u = (u + jnp.uint32(0x7FFF) + ((u >> jnp.uint32(16)) & jnp.uint32(1)))
            return plsc.bitcast(u & jnp.uint32(0xFFFF0000), jnp.float32)

        def per_center(i, carry):
            ci = jnp.full((_L,), i, jnp.int32)
            cx = plsc.load_gather(cx_v, [ci])
            cy = plsc.load_gather(cy_v, [ci])
            cz = plsc.load_gather(cz_v, [ci])
            aa = cx * cx + cy * cy + cz * cz
            cxr, cyr, czr = bf16r(cx), bf16r(cy), bf16r(cz)

            grp = 4

            def cond(st):
                j, c1v, c2v = st
                return (j < nch // grp) & jnp.any((c1v < ns1) | (c2v < ns2))

            def body(st):
                j, c1v, c2v = st
                for u in range(grp):
                    base = j * (grp * _L) + u * _L
                    xi = x_v[pl.ds(base, _L)]
                    yi = y_v[pl.ds(base, _L)]
                    zi = z_v[pl.ds(base, _L)]
                    bb = bb_v[pl.ds(base, _L)]
                    ab = cxr * bf16r(xi) + cyr * bf16r(yi) + czr * bf16r(zi)
                    d = (aa + bb) - 2.0 * ab
                    pidx = lane + base
                    w1 = (d <= r1sq) & (c1v < ns1)
                    w2 = (d <= r2sq) & (c2v < ns2)
                    pos1 = plsc.cumsum(w1.astype(jnp.int32))
                    pos2 = plsc.cumsum(w2.astype(jnp.int32))
                    plsc.store_scatter(buf1, [c1v + pos1 - 1], pidx, mask=w1)
                    plsc.store_scatter(buf2, [c2v + pos2 - 1], pidx, mask=w2)
                    c1v = c1v + plsc.all_reduce_population_count(w1)
                    c2v = c2v + plsc.all_reduce_population_count(w2)
                return j + 1, c1v, c2v

            zero_cnt = jnp.zeros((_L,), jnp.int32)
            _, c1, c2 = lax.while_loop(cond, body, (0, zero_cnt, zero_cnt))

            fi1 = plsc.load_gather(buf1, [zero16])
            safe1 = jnp.where(c1 > 0, fi1, zero16)
            v1 = buf1[0:_L]
            idx1 = jnp.where(lane < c1, v1, safe1)
            gx = plsc.load_gather(x_v, [idx1]) - cx
            gy = plsc.load_gather(y_v, [idx1]) - cy
            gz = plsc.load_gather(z_v, [idx1]) - cz
            row = i * ns1 + lane
            plsc.store_scatter(dxyz1_v, [row, zero16], gx)
            plsc.store_scatter(dxyz1_v, [row, zero16 + 1], gy)
            plsc.store_scatter(dxyz1_v, [row, zero16 + 2], gz)
            flat1 = i * ns1
            gidx1_v[flat1 // 128, pl.ds(flat1 % 128, _L)] = idx1 + boff

            fi2 = plsc.load_gather(buf2, [zero16])
            safe2 = jnp.where(c2 > 0, fi2, zero16)
            for h in range(ns2 // _L):
                slot = lane + h * _L
                v2 = buf2[h * _L:(h + 1) * _L]
                idx2 = jnp.where(slot < c2, v2, safe2)
                gx = plsc.load_gather(x_v, [idx2]) - cx
                gy = plsc.load_gather(y_v, [idx2]) - cy
                gz = plsc.load_gather(z_v, [idx2]) - cz
                row = i * ns2 + slot
                plsc.store_scatter(dxyz2_v, [row, zero16], gx)
                plsc.store_scatter(dxyz2_v, [row, zero16 + 1], gy)
                plsc.store_scatter(dxyz2_v, [row, zero16 + 2], gz)
                flat2 = i * ns2 + h * _L
                gidx2_v[flat2 // 128, pl.ds(flat2 % 128, _L)] = idx2 + boff
            return carry

        lax.fori_loop(0, cpt, per_center, 0)

        pltpu.sync_copy(dxyz1_v, dxyz1_hbm.at[pl.ds(wid * p1t, p1t)])
        pltpu.sync_copy(dxyz2_v, dxyz2_hbm.at[pl.ds(wid * p2t, p2t)])

        for gidx_v, nk, out_hbm, obase in (
                (gidx1_v, g1rows, feat1_hbm, wid * p1t),
                (gidx2_v, g2rows, feat2_hbm, wid * p2t)):
            cp = pltpu.async_copy(feat_hbm.at[gidx_v.at[0]], rows_a, semg)
            for k in range(nk):
                buf = rows_a if k % 2 == 0 else rows_b
                nbuf = rows_b if k % 2 == 0 else rows_a
                cp.wait()
                if k + 1 < nk:
                    cp = pltpu.async_copy(
                        feat_hbm.at[gidx_v.at[k + 1]], nbuf, semg)
                pltpu.sync_copy(buf, out_hbm.at[pl.ds(obase + k * 128, 128)])

    return kern


def _mm_stats_body(feat_ref, dxyz_ref, wf_ref, wx_ref, y_ref, st_ref):
    i = pl.program_id(0)
    x1 = feat_ref[...]
    x2 = dxyz_ref[...]
    col = lax.broadcasted_iota(jnp.int32, x2.shape, 1)
    x2 = jnp.where(col < 3, x2, 0.0)
    y = jnp.dot(x1, wf_ref[...], preferred_element_type=jnp.float32)
    y = y + jnp.dot(x2, wx_ref[...], preferred_element_type=jnp.float32)
    y_ref[...] = y

    @pl.when(i == 0)
    def _init():
        st_ref[...] = jnp.zeros_like(st_ref)

    st_ref[0:1, :] += jnp.sum(y, axis=0, keepdims=True)
    st_ref[1:2, :] += jnp.sum(y * y, axis=0, keepdims=True)


def _mm_stats_call(feat, dxyz, wf, wx, rb):
    p, cin = feat.shape
    c1 = wf.shape[1]
    return pl.pallas_call(
        _mm_stats_body,
        grid=(p // rb,),
        in_specs=[
            pl.BlockSpec((rb, cin), lambda i: (i, 0)),
            pl.BlockSpec((rb, 8), lambda i: (i, 0)),
            pl.BlockSpec((cin, c1), lambda i: (0, 0)),
            pl.BlockSpec((8, c1), lambda i: (0, 0)),
        ],
        out_specs=[
            pl.BlockSpec((rb, c1), lambda i: (i, 0)),
            pl.BlockSpec((8, c1), lambda i: (0, 0)),
        ],
        out_shape=[
            jax.ShapeDtypeStruct((p, c1), jnp.float32),
            jax.ShapeDtypeStruct((8, c1), jnp.float32),
        ],
    )(feat, dxyz, wf, wx)


def _bn_mm_body(y_ref, a_ref, c_ref, w_ref, o_ref, st_ref):
    i = pl.program_id(0)
    a = a_ref[0:1, :]
    c = c_ref[0:1, :]
    x = jnp.maximum(y_ref[...] * a + c, 0.0)
    y = jnp.dot(x, w_ref[...], preferred_element_type=jnp.float32)
    o_ref[...] = y

    @pl.when(i == 0)
    def _init():
        st_ref[...] = jnp.zeros_like(st_ref)

    st_ref[0:1, :] += jnp.sum(y, axis=0, keepdims=True)
    st_ref[1:2, :] += jnp.sum(y * y, axis=0, keepdims=True)


def _bn_mm_call(y1, a, c, w2, rb):
    p, c1 = y1.shape
    c2 = w2.shape[1]
    return pl.pallas_call(
        _bn_mm_body,
        grid=(p // rb,),
        in_specs=[
            pl.BlockSpec((rb, c1), lambda i: (i, 0)),
            pl.BlockSpec((8, c1), lambda i: (0, 0)),
            pl.BlockSpec((8, c1), lambda i: (0, 0)),
            pl.BlockSpec((c1, c2), lambda i: (0, 0)),
        ],
        out_specs=[
            pl.BlockSpec((rb, c2), lambda i: (i, 0)),
            pl.BlockSpec((8, c2), lambda i: (0, 0)),
        ],
        out_shape=[
            jax.ShapeDtypeStruct((p, c2), jnp.float32),
            jax.ShapeDtypeStruct((8, c2), jnp.float32),
        ],
    )(y1, a, c, w2)


def _bn_max_body(y_ref, a_ref, c_ref, o_ref):
    a = jnp.reshape(a_ref[0:1, :], (1, 1, -1))
    c = jnp.reshape(c_ref[0:1, :], (1, 1, -1))
    x = jnp.maximum(y_ref[...] * a + c, 0.0)
    o_ref[...] = jnp.max(x, axis=1)


def _bn_max_call(y3d, a, c, sb):
    ncen, ns, c2 = y3d.shape
    return pl.pallas_call(
        _bn_max_body,
        grid=(ncen // sb,),
        in_specs=[
            pl.BlockSpec((sb, ns, c2), lambda i: (i, 0, 0)),
            pl.BlockSpec((8, c2), lambda i: (0, 0)),
            pl.BlockSpec((8, c2), lambda i: (0, 0)),
        ],
        out_specs=pl.BlockSpec((sb, c2), lambda i: (i, 0)),
        out_shape=jax.ShapeDtypeStruct((ncen, c2), jnp.float32),
    )(y3d, a, c)


def _bn_coeffs(st, count, gamma, beta):
    mu = st[0] / count
    var = st[1] / count - mu * mu
    a = gamma * lax.rsqrt(var + 1e-5)
    c = beta - mu * a
    a8 = jnp.broadcast_to(a[None, :], (8, a.shape[0]))
    c8 = jnp.broadcast_to(c[None, :], (8, c.shape[0]))
    return a8, c8


def kernel(xyz, features, inds, params):
    B, N, _ = xyz.shape
    C = features.shape[1]
    S = inds.shape[1]

    xyz_t = jnp.transpose(xyz, (0, 2, 1)).reshape(B * 3, N)
    inds_flat = inds.reshape(B * S).astype(jnp.int32)
    feat_rows = jnp.transpose(features, (0, 2, 1)).reshape(B * N, C)

    sck = _sc_group_kernel(B, N, S, C)
    nxyz_flat, dxyz1, dxyz2, feat1, feat2 = sck(xyz_t, inds_flat, feat_rows)
    new_xyz = nxyz_flat.reshape(B, S, 3)

    outs = []
    for scale, (ns, dxyz, feat) in enumerate(
            ((_NSAMP[0], dxyz1, feat1), (_NSAMP[1], dxyz2, feat2))):
        (w1, g1, b1), (w2, g2, b2) = params[scale]
        p = B * S * ns
        wf = jnp.transpose(w1[:, 3:])
        wx = jnp.zeros((8, w1.shape[0]), jnp.float32)
        wx = wx.at[0:3].set(jnp.transpose(w1[:, 0:3]))
        y1, st1 = _mm_stats_call(feat, dxyz, wf, wx, rb=4096)
        a1, c1 = _bn_coeffs(st1, p, g1, b1)
        y2, st2 = _bn_mm_call(y1, a1, c1, jnp.transpose(w2), rb=4096)
        a2, c2 = _bn_coeffs(st2, p, g2, b2)
        o = _bn_max_call(y2.reshape(B * S, ns, -1), a2, c2, sb=256)
        outs.append(jnp.transpose(o.reshape(B, S, -1), (0, 2, 1)))

    return new_xyz, jnp.concatenate(outs, axis=1), inds


# speedup vs baseline: 1.1816x; 1.0923x over previous
import functools

import jax
import jax.numpy as jnp
import numpy as np
from jax import lax
from jax.experimental import pallas as pl
from jax.experimental.pallas import tpu as pltpu
from jax.experimental.pallas import tpu_sc as plsc

_RADII = (0.2, 0.4)
_NSAMP = (16, 32)
_NC, _NSUB, _L = 2, 16, 16
_NW = _NC * _NSUB


@functools.lru_cache(maxsize=None)
def _sc_group_kernel(B, N, S, C):
    n_centers = B * S
    cpt = n_centers // _NW
    tiles_per_b = S // cpt
    ns1, ns2 = _NSAMP
    r1sq = np.float32(_RADII[0] * _RADII[0])
    r2sq = np.float32(_RADII[1] * _RADII[1])
    nch = N // _L
    p1t, p2t = cpt * ns1, cpt * ns2
    g1rows, g2rows = p1t // 128, p2t // 128

    mesh = plsc.VectorSubcoreMesh(
        core_axis_name="c", subcore_axis_name="s",
        num_cores=_NC, num_subcores=_NSUB)

    out_type = (
        jax.ShapeDtypeStruct((n_centers * 3,), jnp.float32),
        jax.ShapeDtypeStruct((n_centers * ns1, 8), jnp.float32),
        jax.ShapeDtypeStruct((n_centers * ns2, 8), jnp.float32),
        jax.ShapeDtypeStruct((n_centers * ns1, C), jnp.float32),
        jax.ShapeDtypeStruct((n_centers * ns2, C), jnp.float32),
    )
    scratch = [
        pltpu.VMEM((N,), jnp.float32),
        pltpu.VMEM((N,), jnp.float32),
        pltpu.VMEM((N,), jnp.float32),
        pltpu.VMEM((cpt,), jnp.int32),
        pltpu.VMEM((cpt,), jnp.float32),
        pltpu.VMEM((cpt,), jnp.float32),
        pltpu.VMEM((cpt,), jnp.float32),
        pltpu.VMEM((cpt * 3,), jnp.float32),
        pltpu.VMEM((4, ns1 + _L), jnp.int32),
        pltpu.VMEM((4, ns2 + _L), jnp.int32),
        pltpu.VMEM((p1t, 8), jnp.float32),
        pltpu.VMEM((p2t, 8), jnp.float32),
        pltpu.VMEM((g1rows, 128), jnp.int32),
        pltpu.VMEM((g2rows, 128), jnp.int32),
        pltpu.VMEM((128, C), jnp.float32),
        pltpu.VMEM((128, C), jnp.float32),
        pltpu.VMEM((N,), jnp.float32),
        pltpu.SemaphoreType.DMA,
    ]

    @functools.partial(
        pl.kernel, out_type=out_type, mesh=mesh, scratch_types=scratch,
        compiler_params=pltpu.CompilerParams(
            needs_layout_passes=False, use_tc_tiling_on_sc=False))
    def kern(xyz_hbm, inds_hbm, feat_hbm,
             nxyz_hbm, dxyz1_hbm, dxyz2_hbm, feat1_hbm, feat2_hbm,
             x_v, y_v, z_v, inds_v, cx_v, cy_v, cz_v, nxyz_v,
             buf1, buf2, dxyz1_v, dxyz2_v, gidx1_v, gidx2_v,
             rows_a, rows_b, bb_v, semg):
        wid = lax.axis_index("s") * _NC + lax.axis_index("c")
        b = wid // tiles_per_b
        boff = b * N

        pltpu.sync_copy(xyz_hbm.at[b * 3 + 0], x_v)
        pltpu.sync_copy(xyz_hbm.at[b * 3 + 1], y_v)
        pltpu.sync_copy(xyz_hbm.at[b * 3 + 2], z_v)
        pltpu.sync_copy(inds_hbm.at[pl.ds(wid * cpt, cpt)], inds_v)

        lane = lax.iota(jnp.int32, _L)
        zero16 = jnp.zeros((_L,), jnp.int32)

        for c in range(cpt // _L):
            iv = inds_v[pl.ds(c * _L, _L)]
            gx = plsc.load_gather(x_v, [iv])
            gy = plsc.load_gather(y_v, [iv])
            gz = plsc.load_gather(z_v, [iv])
            cx_v[pl.ds(c * _L, _L)] = gx
            cy_v[pl.ds(c * _L, _L)] = gy
            cz_v[pl.ds(c * _L, _L)] = gz
            pos = (lane + c * _L) * 3
            plsc.store_scatter(nxyz_v, [pos], gx)
            plsc.store_scatter(nxyz_v, [pos + 1], gy)
            plsc.store_scatter(nxyz_v, [pos + 2], gz)
        pltpu.sync_copy(nxyz_v, nxyz_hbm.at[pl.ds(wid * cpt * 3, cpt * 3)])

        def precompute_bb(j, carry):
            base = j * _L
            xi = x_v[pl.ds(base, _L)]
            yi = y_v[pl.ds(base, _L)]
            zi = z_v[pl.ds(base, _L)]
            bb_v[pl.ds(base, _L)] = xi * xi + yi * yi + zi * zi
            return carry

        lax.fori_loop(0, nch, precompute_bb, 0)

        def bf16r(x):
            u = plsc.bitcast(x, jnp.uint32)
            u = (u + jnp.uint32(0x7FFF) + ((u >> jnp.uint32(16)) & jnp.uint32(1)))
            return plsc.bitcast(u & jnp.uint32(0xFFFF0000), jnp.float32)

        ncg = 4
        grp = 4

        def per_group(g, carry):
            cs, aas, rnds = [], [], []
            for t in range(ncg):
                ci = jnp.full((_L,), g * ncg + t, jnp.int32)
                cx = plsc.load_gather(cx_v, [ci])
                cy = plsc.load_gather(cy_v, [ci])
                cz = plsc.load_gather(cz_v, [ci])
                cs.append((cx, cy, cz))
                aas.append(cx * cx + cy * cy + cz * cz)
                rnds.append((bf16r(cx), bf16r(cy), bf16r(cz)))

            def cond(st):
                j = st[0]
                c1s, c2s = st[1], st[2]
                live = (c1s[0] < ns1) | (c2s[0] < ns2)
                for t in range(1, ncg):
                    live = live | (c1s[t] < ns1) | (c2s[t] < ns2)
                return (j < nch // grp) & jnp.any(live)

            def body(st):
                j = st[0]
                c1s, c2s = list(st[1]), list(st[2])
                for u in range(grp):
                    base = j * (grp * _L) + u * _L
                    xr = bf16r(x_v[pl.ds(base, _L)])
                    yr = bf16r(y_v[pl.ds(base, _L)])
                    zr = bf16r(z_v[pl.ds(base, _L)])
                    bb = bb_v[pl.ds(base, _L)]
                    pidx = lane + base
                    for t in range(ncg):
                        cxr, cyr, czr = rnds[t]
                        ab = cxr * xr + cyr * yr + czr * zr
                        d = (aas[t] + bb) - 2.0 * ab
                        w1 = (d <= r1sq) & (c1s[t] < ns1)
                        w2 = (d <= r2sq) & (c2s[t] < ns2)
                        pos1 = plsc.cumsum(w1.astype(jnp.int32))
                        pos2 = plsc.cumsum(w2.astype(jnp.int32))
                        plsc.store_scatter(
                            buf1.at[t], [c1s[t] + pos1 - 1], pidx, mask=w1)
                        plsc.store_scatter(
                            buf2.at[t], [c2s[t] + pos2 - 1], pidx, mask=w2)
                        c1s[t] = c1s[t] + plsc.all_reduce_population_count(w1)
                        c2s[t] = c2s[t] + plsc.all_reduce_population_count(w2)
                return j + 1, tuple(c1s), tuple(c2s)

            zc = jnp.zeros((_L,), jnp.int32)
            _, c1s, c2s = lax.while_loop(
                cond, body, (0, (zc,) * ncg, (zc,) * ncg))

            for t in range(ncg):
                i = g * ncg + t
                cx, cy, cz = cs[t]
                c1, c2 = c1s[t], c2s[t]
                fi1 = plsc.load_gather(buf1, [zero16 + t, zero16])
                safe1 = jnp.where(c1 > 0, fi1, zero16)
                v1 = buf1[t, 0:_L]
                idx1 = jnp.where(lane < c1, v1, safe1)
                gx = plsc.load_gather(x_v, [idx1]) - cx
                gy = plsc.load_gather(y_v, [idx1]) - cy
                gz = plsc.load_gather(z_v, [idx1]) - cz
                row = i * ns1 + lane
                plsc.store_scatter(dxyz1_v, [row, zero16], gx)
                plsc.store_scatter(dxyz1_v, [row, zero16 + 1], gy)
                plsc.store_scatter(dxyz1_v, [row, zero16 + 2], gz)
                flat1 = i * ns1
                gidx1_v[flat1 // 128, pl.ds(flat1 % 128, _L)] = idx1 + boff

                fi2 = plsc.load_gather(buf2, [zero16 + t, zero16])
                safe2 = jnp.where(c2 > 0, fi2, zero16)
                for h in range(ns2 // _L):
                    slot = lane + h * _L
                    v2 = buf2[t, h * _L:(h + 1) * _L]
                    idx2 = jnp.where(slot < c2, v2, safe2)
                    gx = plsc.load_gather(x_v, [idx2]) - cx
                    gy = plsc.load_gather(y_v, [idx2]) - cy
                    gz = plsc.load_gather(z_v, [idx2]) - cz
                    row = i * ns2 + slot
                    plsc.store_scatter(dxyz2_v, [row, zero16], gx)
                    plsc.store_scatter(dxyz2_v, [row, zero16 + 1], gy)
                    plsc.store_scatter(dxyz2_v, [row, zero16 + 2], gz)
                    flat2 = i * ns2 + h * _L
                    gidx2_v[flat2 // 128, pl.ds(flat2 % 128, _L)] = idx2 + boff
            return carry

        lax.fori_loop(0, cpt // ncg, per_group, 0)

        pltpu.sync_copy(dxyz1_v, dxyz1_hbm.at[pl.ds(wid * p1t, p1t)])
        pltpu.sync_copy(dxyz2_v, dxyz2_hbm.at[pl.ds(wid * p2t, p2t)])

        for gidx_v, nk, out_hbm, obase in (
                (gidx1_v, g1rows, feat1_hbm, wid * p1t),
                (gidx2_v, g2rows, feat2_hbm, wid * p2t)):
            cp = pltpu.async_copy(feat_hbm.at[gidx_v.at[0]], rows_a, semg)
            for k in range(nk):
                buf = rows_a if k % 2 == 0 else rows_b
                nbuf = rows_b if k % 2 == 0 else rows_a
                cp.wait()
                if k + 1 < nk:
                    cp = pltpu.async_copy(
                        feat_hbm.at[gidx_v.at[k + 1]], nbuf, semg)
                pltpu.sync_copy(buf, out_hbm.at[pl.ds(obase + k * 128, 128)])

    return kern


def _mm_stats_body(feat_ref, dxyz_ref, wf_ref, wx_ref, y_ref, st_ref):
    i = pl.program_id(0)
    x1 = feat_ref[...]
    x2 = dxyz_ref[...]
    col = lax.broadcasted_iota(jnp.int32, x2.shape, 1)
    x2 = jnp.where(col < 3, x2, 0.0)
    y = jnp.dot(x1, wf_ref[...], preferred_element_type=jnp.float32)
    y = y + jnp.dot(x2, wx_ref[...], preferred_element_type=jnp.float32)
    y_ref[...] = y

    @pl.when(i == 0)
    def _init():
        st_ref[...] = jnp.zeros_like(st_ref)

    st_ref[0:1, :] += jnp.sum(y, axis=0, keepdims=True)
    st_ref[1:2, :] += jnp.sum(y * y, axis=0, keepdims=True)


def _mm_stats_call(feat, dxyz, wf, wx, rb):
    p, cin = feat.shape
    c1 = wf.shape[1]
    return pl.pallas_call(
        _mm_stats_body,
        grid=(p // rb,),
        in_specs=[
            pl.BlockSpec((rb, cin), lambda i: (i, 0)),
            pl.BlockSpec((rb, 8), lambda i: (i, 0)),
            pl.BlockSpec((cin, c1), lambda i: (0, 0)),
            pl.BlockSpec((8, c1), lambda i: (0, 0)),
        ],
        out_specs=[
            pl.BlockSpec((rb, c1), lambda i: (i, 0)),
            pl.BlockSpec((8, c1), lambda i: (0, 0)),
        ],
        out_shape=[
            jax.ShapeDtypeStruct((p, c1), jnp.float32),
            jax.ShapeDtypeStruct((8, c1), jnp.float32),
        ],
    )(feat, dxyz, wf, wx)


def _bn_mm_body(y_ref, a_ref, c_ref, w_ref, o_ref, st_ref):
    i = pl.program_id(0)
    a = a_ref[0:1, :]
    c = c_ref[0:1, :]
    x = jnp.maximum(y_ref[...] * a + c, 0.0)
    y = jnp.dot(x, w_ref[...], preferred_element_type=jnp.float32)
    o_ref[...] = y

    @pl.when(i == 0)
    def _init():
        st_ref[...] = jnp.zeros_like(st_ref)

    st_ref[0:1, :] += jnp.sum(y, axis=0, keepdims=True)
    st_ref[1:2, :] += jnp.sum(y * y, axis=0, keepdims=True)


def _bn_mm_call(y1, a, c, w2, rb):
    p, c1 = y1.shape
    c2 = w2.shape[1]
    return pl.pallas_call(
        _bn_mm_body,
        grid=(p // rb,),
        in_specs=[
            pl.BlockSpec((rb, c1), lambda i: (i, 0)),
            pl.BlockSpec((8, c1), lambda i: (0, 0)),
            pl.BlockSpec((8, c1), lambda i: (0, 0)),
            pl.BlockSpec((c1, c2), lambda i: (0, 0)),
        ],
        out_specs=[
            pl.BlockSpec((rb, c2), lambda i: (i, 0)),
            pl.BlockSpec((8, c2), lambda i: (0, 0)),
        ],
        out_shape=[
            jax.ShapeDtypeStruct((p, c2), jnp.float32),
            jax.ShapeDtypeStruct((8, c2), jnp.float32),
        ],
    )(y1, a, c, w2)


def _bn_max_body(y_ref, a_ref, c_ref, o_ref):
    a = jnp.reshape(a_ref[0:1, :], (1, 1, -1))
    c = jnp.reshape(c_ref[0:1, :], (1, 1, -1))
    x = jnp.maximum(y_ref[...] * a + c, 0.0)
    o_ref[...] = jnp.max(x, axis=1)


def _bn_max_call(y3d, a, c, sb):
    ncen, ns, c2 = y3d.shape
    return pl.pallas_call(
        _bn_max_body,
        grid=(ncen // sb,),
        in_specs=[
            pl.BlockSpec((sb, ns, c2), lambda i: (i, 0, 0)),
            pl.BlockSpec((8, c2), lambda i: (0, 0)),
            pl.BlockSpec((8, c2), lambda i: (0, 0)),
        ],
        out_specs=pl.BlockSpec((sb, c2), lambda i: (i, 0)),
        out_shape=jax.ShapeDtypeStruct((ncen, c2), jnp.float32),
    )(y3d, a, c)


def _bn_coeffs(st, count, gamma, beta):
    mu = st[0] / count
    var = st[1] / count - mu * mu
    a = gamma * lax.rsqrt(var + 1e-5)
    c = beta - mu * a
    a8 = jnp.broadcast_to(a[None, :], (8, a.shape[0]))
    c8 = jnp.broadcast_to(c[None, :], (8, c.shape[0]))
    return a8, c8


def kernel(xyz, features, inds, params):
    B, N, _ = xyz.shape
    C = features.shape[1]
    S = inds.shape[1]

    xyz_t = jnp.transpose(xyz, (0, 2, 1)).reshape(B * 3, N)
    inds_flat = inds.reshape(B * S).astype(jnp.int32)
    feat_rows = jnp.transpose(features, (0, 2, 1)).reshape(B * N, C)

    sck = _sc_group_kernel(B, N, S, C)
    nxyz_flat, dxyz1, dxyz2, feat1, feat2 = sck(xyz_t, inds_flat, feat_rows)
    new_xyz = nxyz_flat.reshape(B, S, 3)

    outs = []
    for scale, (ns, dxyz, feat) in enumerate(
            ((_NSAMP[0], dxyz1, feat1), (_NSAMP[1], dxyz2, feat2))):
        (w1, g1, b1), (w2, g2, b2) = params[scale]
        p = B * S * ns
        wf = jnp.transpose(w1[:, 3:])
        wx = jnp.zeros((8, w1.shape[0]), jnp.float32)
        wx = wx.at[0:3].set(jnp.transpose(w1[:, 0:3]))
        y1, st1 = _mm_stats_call(feat, dxyz, wf, wx, rb=4096)
        a1, c1 = _bn_coeffs(st1, p, g1, b1)
        y2, st2 = _bn_mm_call(y1, a1, c1, jnp.transpose(w2), rb=4096)
        a2, c2 = _bn_coeffs(st2, p, g2, b2)
        o = _bn_max_call(y2.reshape(B * S, ns, -1), a2, c2, sb=256)
        outs.append(jnp.transpose(o.reshape(B, S, -1), (0, 2, 1)))

    return new_xyz, jnp.concatenate(outs, axis=1), inds
